# Initial kernel scaffold; baseline (speedup 1.0000x reference)
import jax
import jax.numpy as jnp
from jax.experimental import pallas as pl


def kernel(state, edge_index, deterministic, conv_W, conv_b, lin1_W, lin1_b, lin2_W, lin2_b, dir_W, dir_b):
    raise NotImplementedError("write your pallas kernel here")


# speedup vs baseline: 15.6569x; 15.6569x over previous
import functools

import jax
import jax.numpy as jnp
from jax import lax
from jax.experimental import pallas as pl
from jax.experimental.pallas import tpu as pltpu
from jax.experimental.pallas import tpu_sc as plsc

N = 10000
E = 160000
D = 256
HALF = 128
ACT = 8

NPAD = 10240
ROWS_PER_TILE = NPAD // 16

AGG_CH = 125
AGG_ROWS = E // AGG_CH
AGG_TROWS = AGG_ROWS // 16
DEG_TROWS = AGG_ROWS // 32


def _sc_mesh():
    return plsc.VectorSubcoreMesh(core_axis_name="c", subcore_axis_name="s")


def _deg_body(dst_hbm, ones_hbm, zeros_hbm, deg0_hbm, deg1_hbm,
              dstv, onesv, accd):
    c = lax.axis_index("c")
    s = lax.axis_index("s")
    pltpu.sync_copy(zeros_hbm.at[pl.ds(s * 640, 640)],
                    accd.at[pl.ds(s * 640, 640)])
    pltpu.sync_copy(ones_hbm.at[pl.ds(0, AGG_CH)], onesv)
    r0 = c * (AGG_ROWS // 2) + s * DEG_TROWS
    pltpu.sync_copy(dst_hbm.at[pl.ds(r0, DEG_TROWS)], dstv)
    plsc.subcore_barrier()

    def body(j, carry):
        pltpu.sync_copy(onesv, accd.at[dstv.at[j]], add=True)
        return carry

    lax.fori_loop(0, DEG_TROWS, body, 0)
    plsc.subcore_barrier()

    @pl.when(c == 0)
    def _():
        pltpu.sync_copy(accd.at[pl.ds(s * 640, 640)],
                        deg0_hbm.at[pl.ds(s * 640, 640)])

    @pl.when(c == 1)
    def _():
        pltpu.sync_copy(accd.at[pl.ds(s * 640, 640)],
                        deg1_hbm.at[pl.ds(s * 640, 640)])


@jax.jit
def _sc_degree(dst_a, ones_hbm, zeros_deg):
    f = functools.partial(
        pl.kernel,
        mesh=_sc_mesh(),
        out_type=[jax.ShapeDtypeStruct((NPAD,), jnp.float32),
                  jax.ShapeDtypeStruct((NPAD,), jnp.float32)],
        scratch_types=[
            pltpu.VMEM((DEG_TROWS, AGG_CH), jnp.int32),
            pltpu.VMEM((AGG_CH,), jnp.float32),
            pltpu.VMEM_SHARED((NPAD,), jnp.float32),
        ],
    )
    return f(_deg_body)(dst_a, ones_hbm, zeros_deg)


def _agg_body(y0_hbm, y1_hbm, src_hbm, dst_hbm, zeros_hbm,
              agg0_hbm, agg1_hbm, srcv, dstv, rows, acc, sem):
    c = lax.axis_index("c")
    s = lax.axis_index("s")
    pltpu.sync_copy(zeros_hbm.at[pl.ds(s * ROWS_PER_TILE, ROWS_PER_TILE)],
                    acc.at[pl.ds(s * ROWS_PER_TILE, ROWS_PER_TILE)])
    r0 = s * AGG_TROWS
    pltpu.sync_copy(src_hbm.at[pl.ds(r0, AGG_TROWS)], srcv)
    pltpu.sync_copy(dst_hbm.at[pl.ds(r0, AGG_TROWS)], dstv)
    plsc.subcore_barrier()

    def make_loop(ytab):
        def body(j, carry):
            pltpu.async_copy(ytab.at[srcv.at[j]], rows, sem).wait()
            pltpu.sync_copy(rows, acc.at[dstv.at[j]], add=True)
            return carry
        return body

    @pl.when(c == 0)
    def _():
        lax.fori_loop(0, AGG_TROWS, make_loop(y0_hbm), 0)

    @pl.when(c == 1)
    def _():
        lax.fori_loop(0, AGG_TROWS, make_loop(y1_hbm), 0)

    plsc.subcore_barrier()

    @pl.when(c == 0)
    def _():
        pltpu.sync_copy(acc.at[pl.ds(s * ROWS_PER_TILE, ROWS_PER_TILE)],
                        agg0_hbm.at[pl.ds(s * ROWS_PER_TILE, ROWS_PER_TILE)])

    @pl.when(c == 1)
    def _():
        pltpu.sync_copy(acc.at[pl.ds(s * ROWS_PER_TILE, ROWS_PER_TILE)],
                        agg1_hbm.at[pl.ds(s * ROWS_PER_TILE, ROWS_PER_TILE)])


@jax.jit
def _sc_aggregate(y0, y1, src_a, dst_a, zeros_feat):
    f = functools.partial(
        pl.kernel,
        mesh=_sc_mesh(),
        out_type=[jax.ShapeDtypeStruct((NPAD, HALF), jnp.float32),
                  jax.ShapeDtypeStruct((NPAD, HALF), jnp.float32)],
        scratch_types=[
            pltpu.VMEM((AGG_TROWS, AGG_CH), jnp.int32),
            pltpu.VMEM((AGG_TROWS, AGG_CH), jnp.int32),
            pltpu.VMEM((AGG_CH, HALF), jnp.float32),
            pltpu.VMEM_SHARED((NPAD, HALF), jnp.float32),
            pltpu.SemaphoreType.DMA,
        ],
    )
    return f(_agg_body)(y0, y1, src_a, dst_a, zeros_feat)


BLK = 1000


def _pre_body(x_ref, wt_ref, d0_ref, d1_ref, y0_ref, y1_ref, dinv_ref):
    deg = d0_ref[...] + d1_ref[...] + 1.0
    dinv = lax.rsqrt(deg)
    xw = jnp.dot(x_ref[...], wt_ref[...], preferred_element_type=jnp.float32)
    y = xw * dinv
    y0_ref[...] = y[:, :HALF]
    y1_ref[...] = y[:, HALF:]
    dinv_ref[...] = dinv


@jax.jit
def _tc_pre(state, conv_Wt, deg0, deg1):
    nblk = N // BLK
    return pl.pallas_call(
        _pre_body,
        grid=(nblk,),
        in_specs=[
            pl.BlockSpec((BLK, D), lambda i: (i, 0)),
            pl.BlockSpec((D, D), lambda i: (0, 0)),
            pl.BlockSpec((BLK, 1), lambda i: (i, 0)),
            pl.BlockSpec((BLK, 1), lambda i: (i, 0)),
        ],
        out_specs=[
            pl.BlockSpec((BLK, HALF), lambda i: (i, 0)),
            pl.BlockSpec((BLK, HALF), lambda i: (i, 0)),
            pl.BlockSpec((BLK, 1), lambda i: (i, 0)),
        ],
        out_shape=[
            jax.ShapeDtypeStruct((N, HALF), jnp.float32),
            jax.ShapeDtypeStruct((N, HALF), jnp.float32),
            jax.ShapeDtypeStruct((N, 1), jnp.float32),
        ],
    )(state, conv_Wt, deg0, deg1)


def _mlp_body(a0_ref, a1_ref, y0_ref, y1_ref, dinv_ref, st_ref, cb_ref,
              w1_ref, b1_ref, w2_ref, b2_ref, dw_ref, db_ref, conc_ref):
    agg = jnp.concatenate([a0_ref[...], a1_ref[...]], axis=1)
    y = jnp.concatenate([y0_ref[...], y1_ref[...]], axis=1)
    dinv = dinv_ref[...]
    x = jnp.maximum(dinv * (agg + y) + cb_ref[...], 0.0) + st_ref[...]
    h = jnp.dot(x, w1_ref[...], preferred_element_type=jnp.float32) + b1_ref[...]
    h = jnp.where(h > 0, h, 0.01 * h)
    h = jnp.dot(h, w2_ref[...], preferred_element_type=jnp.float32) + b2_ref[...]
    h = jnp.where(h > 0, h, 0.01 * h)
    z = jnp.sum(h * dw_ref[...], axis=1, keepdims=True) + db_ref[...]
    conc_ref[...] = jnp.maximum(z, 0.0) + jnp.log1p(jnp.exp(-jnp.abs(z)))


@jax.jit
def _tc_mlp(agg0, agg1, y0, y1, dinv, state, conv_b, l1wt, l1b, l2wt, l2b,
            dirw, dirb):
    nblk = N // BLK
    blk = lambda r, cdim: pl.BlockSpec((BLK, cdim), lambda i: (i, 0))
    full = lambda rdim, cdim: pl.BlockSpec((rdim, cdim), lambda i: (0, 0))
    return pl.pallas_call(
        _mlp_body,
        grid=(nblk,),
        in_specs=[
            blk(BLK, HALF), blk(BLK, HALF), blk(BLK, HALF), blk(BLK, HALF),
            blk(BLK, 1), blk(BLK, D),
            full(1, D), full(D, D), full(1, D), full(D, D), full(1, D),
            full(1, D), full(1, 1),
        ],
        out_specs=blk(BLK, 1),
        out_shape=jax.ShapeDtypeStruct((N, 1), jnp.float32),
    )(agg0, agg1, y0, y1, dinv, state, conv_b, l1wt, l1b, l2wt, l2b, dirw,
      dirb)


def _norm_body(c_ref, o_ref):
    c = c_ref[...]
    o_ref[...] = c / (jnp.sum(c) + 1e-20)


@jax.jit
def _tc_norm(conc):
    return pl.pallas_call(
        _norm_body,
        out_shape=jax.ShapeDtypeStruct((N, 1), jnp.float32),
    )(conc)


def kernel(state, edge_index, deterministic, conv_W, conv_b, lin1_W, lin1_b,
           lin2_W, lin2_b, dir_W, dir_b):
    src = edge_index[0]
    dst = edge_index[1]
    src_agg = src.reshape(AGG_ROWS, AGG_CH)
    dst_agg = dst.reshape(AGG_ROWS, AGG_CH)
    ones128 = jnp.ones((128,), jnp.float32)
    zeros_deg = jnp.zeros((NPAD,), jnp.float32)
    zeros_feat = jnp.zeros((NPAD, HALF), jnp.float32)

    deg0, deg1 = _sc_degree(dst_agg, ones128, zeros_deg)
    y0, y1, dinv = _tc_pre(state, conv_W.T, deg0[:N].reshape(N, 1),
                           deg1[:N].reshape(N, 1))
    agg0, agg1 = _sc_aggregate(y0, y1, src_agg, dst_agg, zeros_feat)
    conc = _tc_mlp(agg0, agg1, y0, y1, dinv, state,
                   conv_b.reshape(1, D), lin1_W.T, lin1_b.reshape(1, D),
                   lin2_W.T, lin2_b.reshape(1, D), dir_W.reshape(1, D),
                   dir_b.reshape(1, 1))
    action = _tc_norm(conc)
    return action.reshape(N // ACT, ACT)


# speedup vs baseline: 18.7229x; 1.1958x over previous
import functools

import jax
import jax.numpy as jnp
from jax import lax
from jax.experimental import pallas as pl
from jax.experimental.pallas import tpu as pltpu
from jax.experimental.pallas import tpu_sc as plsc

N = 10000
E = 160000
D = 256
HALF = 128
ACT = 8

NPAD = 10240
ROWS_PER_TILE = NPAD // 16

AGG_CH = 125
AGG_ROWS = E // AGG_CH
AGG_TROWS = AGG_ROWS // 16
DEG_TROWS = AGG_ROWS // 32


def _sc_mesh():
    return plsc.VectorSubcoreMesh(core_axis_name="c", subcore_axis_name="s")


def _deg_body(dst_hbm, ones_hbm, zeros_hbm, deg0_hbm, deg1_hbm,
              dstv, onesv, accd):
    c = lax.axis_index("c")
    s = lax.axis_index("s")
    pltpu.sync_copy(zeros_hbm.at[pl.ds(s * 640, 640)],
                    accd.at[pl.ds(s * 640, 640)])
    pltpu.sync_copy(ones_hbm.at[pl.ds(0, AGG_CH)], onesv)
    r0 = c * (AGG_ROWS // 2) + s * DEG_TROWS
    pltpu.sync_copy(dst_hbm.at[pl.ds(r0, DEG_TROWS)], dstv)
    plsc.subcore_barrier()

    def body(j, carry):
        pltpu.sync_copy(onesv, accd.at[dstv.at[j]], add=True)
        return carry

    lax.fori_loop(0, DEG_TROWS, body, 0)
    plsc.subcore_barrier()

    @pl.when(c == 0)
    def _():
        pltpu.sync_copy(accd.at[pl.ds(s * 640, 640)],
                        deg0_hbm.at[pl.ds(s * 640, 640)])

    @pl.when(c == 1)
    def _():
        pltpu.sync_copy(accd.at[pl.ds(s * 640, 640)],
                        deg1_hbm.at[pl.ds(s * 640, 640)])


@jax.jit
def _sc_degree(dst_a, ones_hbm, zeros_deg):
    f = functools.partial(
        pl.kernel,
        mesh=_sc_mesh(),
        out_type=[jax.ShapeDtypeStruct((NPAD,), jnp.float32),
                  jax.ShapeDtypeStruct((NPAD,), jnp.float32)],
        scratch_types=[
            pltpu.VMEM((DEG_TROWS, AGG_CH), jnp.int32),
            pltpu.VMEM((AGG_CH,), jnp.float32),
            pltpu.VMEM_SHARED((NPAD,), jnp.float32),
        ],
    )
    return f(_deg_body)(dst_a, ones_hbm, zeros_deg)


def _agg_body(y0_hbm, y1_hbm, src_hbm, dst_hbm, zeros_hbm,
              agg0_hbm, agg1_hbm, srcv, dstv, rows0, rows1, acc, sem0, sem1):
    c = lax.axis_index("c")
    s = lax.axis_index("s")
    pltpu.sync_copy(zeros_hbm.at[pl.ds(s * ROWS_PER_TILE, ROWS_PER_TILE)],
                    acc.at[pl.ds(s * ROWS_PER_TILE, ROWS_PER_TILE)])
    r0 = s * AGG_TROWS
    plsc.subcore_barrier()

    HTR = AGG_TROWS // 2

    def make_loop(ytab):
        def start(j, buf, sem):
            pltpu.async_copy(ytab.at[srcv.at[j]], buf, sem)

        def drain(j, buf, sem):
            pltpu.make_async_copy(ytab.at[srcv.at[j]], buf, sem).wait()

        for h in range(2):
            pltpu.sync_copy(src_hbm.at[pl.ds(r0 + h * HTR, HTR)], srcv)
            pltpu.sync_copy(dst_hbm.at[pl.ds(r0 + h * HTR, HTR)], dstv)
            start(0, rows0, sem0)

            def body(k, carry):
                j0 = 2 * k
                j1 = j0 + 1
                drain(j0, rows0, sem0)
                start(j1, rows1, sem1)
                pltpu.sync_copy(rows0, acc.at[dstv.at[j0]], add=True)
                drain(j1, rows1, sem1)

                @pl.when(k < HTR // 2 - 1)
                def _():
                    start(j1 + 1, rows0, sem0)

                pltpu.sync_copy(rows1, acc.at[dstv.at[j1]], add=True)
                return carry

            lax.fori_loop(0, HTR // 2, body, 0)

    @pl.when(c == 0)
    def _():
        make_loop(y0_hbm)

    @pl.when(c == 1)
    def _():
        make_loop(y1_hbm)

    plsc.subcore_barrier()

    @pl.when(c == 0)
    def _():
        pltpu.sync_copy(acc.at[pl.ds(s * ROWS_PER_TILE, ROWS_PER_TILE)],
                        agg0_hbm.at[pl.ds(s * ROWS_PER_TILE, ROWS_PER_TILE)])

    @pl.when(c == 1)
    def _():
        pltpu.sync_copy(acc.at[pl.ds(s * ROWS_PER_TILE, ROWS_PER_TILE)],
                        agg1_hbm.at[pl.ds(s * ROWS_PER_TILE, ROWS_PER_TILE)])


@jax.jit
def _sc_aggregate(y0, y1, src_a, dst_a, zeros_feat):
    f = functools.partial(
        pl.kernel,
        mesh=_sc_mesh(),
        out_type=[jax.ShapeDtypeStruct((NPAD, HALF), jnp.float32),
                  jax.ShapeDtypeStruct((NPAD, HALF), jnp.float32)],
        scratch_types=[
            pltpu.VMEM((AGG_TROWS // 2, AGG_CH), jnp.int32),
            pltpu.VMEM((AGG_TROWS // 2, AGG_CH), jnp.int32),
            pltpu.VMEM((AGG_CH, HALF), jnp.float32),
            pltpu.VMEM((AGG_CH, HALF), jnp.float32),
            pltpu.VMEM_SHARED((NPAD, HALF), jnp.float32),
            pltpu.SemaphoreType.DMA,
            pltpu.SemaphoreType.DMA,
        ],
    )
    return f(_agg_body)(y0, y1, src_a, dst_a, zeros_feat)


BLK = 1000


def _pre_body(x_ref, wt_ref, d0_ref, d1_ref, y0_ref, y1_ref, dinv_ref):
    deg = d0_ref[...] + d1_ref[...] + 1.0
    dinv = lax.rsqrt(deg)
    xw = jnp.dot(x_ref[...], wt_ref[...], preferred_element_type=jnp.float32)
    y = xw * dinv
    y0_ref[...] = y[:, :HALF]
    y1_ref[...] = y[:, HALF:]
    dinv_ref[...] = dinv


@jax.jit
def _tc_pre(state, conv_Wt, deg0, deg1):
    nblk = N // BLK
    return pl.pallas_call(
        _pre_body,
        grid=(nblk,),
        in_specs=[
            pl.BlockSpec((BLK, D), lambda i: (i, 0)),
            pl.BlockSpec((D, D), lambda i: (0, 0)),
            pl.BlockSpec((BLK, 1), lambda i: (i, 0)),
            pl.BlockSpec((BLK, 1), lambda i: (i, 0)),
        ],
        out_specs=[
            pl.BlockSpec((BLK, HALF), lambda i: (i, 0)),
            pl.BlockSpec((BLK, HALF), lambda i: (i, 0)),
            pl.BlockSpec((BLK, 1), lambda i: (i, 0)),
        ],
        out_shape=[
            jax.ShapeDtypeStruct((N, HALF), jnp.float32),
            jax.ShapeDtypeStruct((N, HALF), jnp.float32),
            jax.ShapeDtypeStruct((N, 1), jnp.float32),
        ],
    )(state, conv_Wt, deg0, deg1)


def _mlp_body(a0_ref, a1_ref, y0_ref, y1_ref, dinv_ref, st_ref, cb_ref,
              w1_ref, b1_ref, w2_ref, b2_ref, dw_ref, db_ref, conc_ref):
    agg = jnp.concatenate([a0_ref[...], a1_ref[...]], axis=1)
    y = jnp.concatenate([y0_ref[...], y1_ref[...]], axis=1)
    dinv = dinv_ref[...]
    x = jnp.maximum(dinv * (agg + y) + cb_ref[...], 0.0) + st_ref[...]
    h = jnp.dot(x, w1_ref[...], preferred_element_type=jnp.float32) + b1_ref[...]
    h = jnp.where(h > 0, h, 0.01 * h)
    h = jnp.dot(h, w2_ref[...], preferred_element_type=jnp.float32) + b2_ref[...]
    h = jnp.where(h > 0, h, 0.01 * h)
    z = jnp.sum(h * dw_ref[...], axis=1, keepdims=True) + db_ref[...]
    conc_ref[...] = jnp.maximum(z, 0.0) + jnp.log1p(jnp.exp(-jnp.abs(z)))


@jax.jit
def _tc_mlp(agg0, agg1, y0, y1, dinv, state, conv_b, l1wt, l1b, l2wt, l2b,
            dirw, dirb):
    nblk = N // BLK
    blk = lambda r, cdim: pl.BlockSpec((BLK, cdim), lambda i: (i, 0))
    full = lambda rdim, cdim: pl.BlockSpec((rdim, cdim), lambda i: (0, 0))
    return pl.pallas_call(
        _mlp_body,
        grid=(nblk,),
        in_specs=[
            blk(BLK, HALF), blk(BLK, HALF), blk(BLK, HALF), blk(BLK, HALF),
            blk(BLK, 1), blk(BLK, D),
            full(1, D), full(D, D), full(1, D), full(D, D), full(1, D),
            full(1, D), full(1, 1),
        ],
        out_specs=blk(BLK, 1),
        out_shape=jax.ShapeDtypeStruct((N, 1), jnp.float32),
    )(agg0, agg1, y0, y1, dinv, state, conv_b, l1wt, l1b, l2wt, l2b, dirw,
      dirb)


def _norm_body(c_ref, o_ref):
    c = c_ref[...]
    o_ref[...] = c / (jnp.sum(c) + 1e-20)


@jax.jit
def _tc_norm(conc):
    return pl.pallas_call(
        _norm_body,
        out_shape=jax.ShapeDtypeStruct((N, 1), jnp.float32),
    )(conc)


def kernel(state, edge_index, deterministic, conv_W, conv_b, lin1_W, lin1_b,
           lin2_W, lin2_b, dir_W, dir_b):
    src = edge_index[0]
    dst = edge_index[1]
    src_agg = src.reshape(AGG_ROWS, AGG_CH)
    dst_agg = dst.reshape(AGG_ROWS, AGG_CH)
    ones128 = jnp.ones((128,), jnp.float32)
    zeros_deg = jnp.zeros((NPAD,), jnp.float32)
    zeros_feat = jnp.zeros((NPAD, HALF), jnp.float32)

    deg0, deg1 = _sc_degree(dst_agg, ones128, zeros_deg)
    y0, y1, dinv = _tc_pre(state, conv_W.T, deg0[:N].reshape(N, 1),
                           deg1[:N].reshape(N, 1))
    agg0, agg1 = _sc_aggregate(y0, y1, src_agg, dst_agg, zeros_feat)
    conc = _tc_mlp(agg0, agg1, y0, y1, dinv, state,
                   conv_b.reshape(1, D), lin1_W.T, lin1_b.reshape(1, D),
                   lin2_W.T, lin2_b.reshape(1, D), dir_W.reshape(1, D),
                   dir_b.reshape(1, 1))
    action = _tc_norm(conc)
    return action.reshape(N // ACT, ACT)


# speedup vs baseline: 19.1141x; 1.0209x over previous
import functools

import jax
import jax.numpy as jnp
from jax import lax
from jax.experimental import pallas as pl
from jax.experimental.pallas import tpu as pltpu
from jax.experimental.pallas import tpu_sc as plsc

N = 10000
E = 160000
D = 256
HALF = 128
ACT = 8

NPAD = 10240
ROWS_PER_TILE = NPAD // 16

AGG_CH = 125
AGG_ROWS = E // AGG_CH
AGG_TROWS = AGG_ROWS // 16
DEG_TROWS = AGG_ROWS // 32

ZROWS = 32


def _sc_mesh():
    return plsc.VectorSubcoreMesh(core_axis_name="c", subcore_axis_name="s")


def _deg_body(dst_hbm, ones_hbm, zeros_hbm, deg0_hbm, deg1_hbm,
              dstv, onesv, accd):
    c = lax.axis_index("c")
    s = lax.axis_index("s")
    pltpu.sync_copy(zeros_hbm.at[pl.ds(s * 640, 640)],
                    accd.at[pl.ds(s * 640, 640)])
    pltpu.sync_copy(ones_hbm.at[pl.ds(0, AGG_CH)], onesv)
    r0 = c * (AGG_ROWS // 2) + s * DEG_TROWS
    pltpu.sync_copy(dst_hbm.at[pl.ds(r0, DEG_TROWS)], dstv)
    plsc.subcore_barrier()

    def body(j, carry):
        pltpu.sync_copy(onesv, accd.at[dstv.at[j]], add=True)
        return carry

    lax.fori_loop(0, DEG_TROWS, body, 0)
    plsc.subcore_barrier()

    @pl.when(c == 0)
    def _():
        pltpu.sync_copy(accd.at[pl.ds(s * 640, 640)],
                        deg0_hbm.at[pl.ds(s * 640, 640)])

    @pl.when(c == 1)
    def _():
        pltpu.sync_copy(accd.at[pl.ds(s * 640, 640)],
                        deg1_hbm.at[pl.ds(s * 640, 640)])


@jax.jit
def _sc_degree(dst_a, ones_hbm, zeros_deg):
    f = functools.partial(
        pl.kernel,
        mesh=_sc_mesh(),
        out_type=[jax.ShapeDtypeStruct((NPAD,), jnp.float32),
                  jax.ShapeDtypeStruct((NPAD,), jnp.float32)],
        scratch_types=[
            pltpu.VMEM((DEG_TROWS, AGG_CH), jnp.int32),
            pltpu.VMEM((AGG_CH,), jnp.float32),
            pltpu.VMEM_SHARED((NPAD,), jnp.float32),
        ],
    )
    return f(_deg_body)(dst_a, ones_hbm, zeros_deg)


def _agg_body(y0_hbm, y1_hbm, src_hbm, dst_hbm,
              agg0_hbm, agg1_hbm, srcv, dstv, rows0, rows1, zbuf, acc,
              sem0, sem1):
    c = lax.axis_index("c")
    s = lax.axis_index("s")

    def zstore(r, carry):
        for k in range(HALF // 16):
            zbuf[r, pl.ds(16 * k, 16)] = jnp.zeros((16,), jnp.float32)
        return carry

    lax.fori_loop(0, ZROWS, zstore, 0)
    for m in range(ROWS_PER_TILE // ZROWS):
        pltpu.sync_copy(zbuf,
                        acc.at[pl.ds(s * ROWS_PER_TILE + m * ZROWS, ZROWS)])

    r0 = s * AGG_TROWS
    plsc.subcore_barrier()

    HTR = AGG_TROWS // 2

    def make_loop(ytab):
        def start(j, buf, sem):
            pltpu.async_copy(ytab.at[srcv.at[j]], buf, sem)

        def drain(j, buf, sem):
            pltpu.make_async_copy(ytab.at[srcv.at[j]], buf, sem).wait()

        for h in range(2):
            pltpu.sync_copy(src_hbm.at[pl.ds(r0 + h * HTR, HTR)], srcv)
            pltpu.sync_copy(dst_hbm.at[pl.ds(r0 + h * HTR, HTR)], dstv)
            start(0, rows0, sem0)

            def body(k, carry):
                j0 = 2 * k
                j1 = j0 + 1
                drain(j0, rows0, sem0)
                start(j1, rows1, sem1)
                pltpu.sync_copy(rows0, acc.at[dstv.at[j0]], add=True)
                drain(j1, rows1, sem1)

                @pl.when(k < HTR // 2 - 1)
                def _():
                    start(j1 + 1, rows0, sem0)

                pltpu.sync_copy(rows1, acc.at[dstv.at[j1]], add=True)
                return carry

            lax.fori_loop(0, HTR // 2, body, 0)

    @pl.when(c == 0)
    def _():
        make_loop(y0_hbm)

    @pl.when(c == 1)
    def _():
        make_loop(y1_hbm)

    plsc.subcore_barrier()

    @pl.when(c == 0)
    def _():
        pltpu.sync_copy(acc.at[pl.ds(s * ROWS_PER_TILE, ROWS_PER_TILE)],
                        agg0_hbm.at[pl.ds(s * ROWS_PER_TILE, ROWS_PER_TILE)])

    @pl.when(c == 1)
    def _():
        pltpu.sync_copy(acc.at[pl.ds(s * ROWS_PER_TILE, ROWS_PER_TILE)],
                        agg1_hbm.at[pl.ds(s * ROWS_PER_TILE, ROWS_PER_TILE)])


@jax.jit
def _sc_aggregate(y0, y1, src_a, dst_a):
    f = functools.partial(
        pl.kernel,
        mesh=_sc_mesh(),
        out_type=[jax.ShapeDtypeStruct((NPAD, HALF), jnp.float32),
                  jax.ShapeDtypeStruct((NPAD, HALF), jnp.float32)],
        scratch_types=[
            pltpu.VMEM((AGG_TROWS // 2, AGG_CH), jnp.int32),
            pltpu.VMEM((AGG_TROWS // 2, AGG_CH), jnp.int32),
            pltpu.VMEM((AGG_CH, HALF), jnp.float32),
            pltpu.VMEM((AGG_CH, HALF), jnp.float32),
            pltpu.VMEM((ZROWS, HALF), jnp.float32),
            pltpu.VMEM_SHARED((NPAD, HALF), jnp.float32),
            pltpu.SemaphoreType.DMA,
            pltpu.SemaphoreType.DMA,
        ],
    )
    return f(_agg_body)(y0, y1, src_a, dst_a)


BLK = 1000


def _xw_body(x_ref, wt_ref, xw_ref):
    xw_ref[...] = jnp.dot(x_ref[...], wt_ref[...],
                          preferred_element_type=jnp.float32)


@jax.jit
def _tc_xw(state, conv_Wt):
    return pl.pallas_call(
        _xw_body,
        grid=(N // BLK,),
        in_specs=[
            pl.BlockSpec((BLK, D), lambda i: (i, 0)),
            pl.BlockSpec((D, D), lambda i: (0, 0)),
        ],
        out_specs=pl.BlockSpec((BLK, D), lambda i: (i, 0)),
        out_shape=jax.ShapeDtypeStruct((N, D), jnp.float32),
    )(state, conv_Wt)


def _scale_body(xw_ref, d0_ref, d1_ref, y0_ref, y1_ref, dinv_ref):
    deg = d0_ref[...] + d1_ref[...] + 1.0
    dinv = lax.rsqrt(deg)
    y = xw_ref[...] * dinv
    y0_ref[...] = y[:, :HALF]
    y1_ref[...] = y[:, HALF:]
    dinv_ref[...] = dinv


@jax.jit
def _tc_scale(xw, deg0, deg1):
    return pl.pallas_call(
        _scale_body,
        grid=(N // BLK,),
        in_specs=[
            pl.BlockSpec((BLK, D), lambda i: (i, 0)),
            pl.BlockSpec((BLK, 1), lambda i: (i, 0)),
            pl.BlockSpec((BLK, 1), lambda i: (i, 0)),
        ],
        out_specs=[
            pl.BlockSpec((BLK, HALF), lambda i: (i, 0)),
            pl.BlockSpec((BLK, HALF), lambda i: (i, 0)),
            pl.BlockSpec((BLK, 1), lambda i: (i, 0)),
        ],
        out_shape=[
            jax.ShapeDtypeStruct((N, HALF), jnp.float32),
            jax.ShapeDtypeStruct((N, HALF), jnp.float32),
            jax.ShapeDtypeStruct((N, 1), jnp.float32),
        ],
    )(xw, deg0, deg1)


def _mlp_body(a0_ref, a1_ref, y0_ref, y1_ref, dinv_ref, st_ref, cb_ref,
              w1_ref, b1_ref, w2_ref, b2_ref, dw_ref, db_ref, conc_ref):
    agg = jnp.concatenate([a0_ref[...], a1_ref[...]], axis=1)
    y = jnp.concatenate([y0_ref[...], y1_ref[...]], axis=1)
    dinv = dinv_ref[...]
    x = jnp.maximum(dinv * (agg + y) + cb_ref[...], 0.0) + st_ref[...]
    h = jnp.dot(x, w1_ref[...], preferred_element_type=jnp.float32) + b1_ref[...]
    h = jnp.where(h > 0, h, 0.01 * h)
    h = jnp.dot(h, w2_ref[...], preferred_element_type=jnp.float32) + b2_ref[...]
    h = jnp.where(h > 0, h, 0.01 * h)
    z = jnp.sum(h * dw_ref[...], axis=1, keepdims=True) + db_ref[...]
    conc_ref[...] = jnp.maximum(z, 0.0) + jnp.log1p(jnp.exp(-jnp.abs(z)))


@jax.jit
def _tc_mlp(agg0, agg1, y0, y1, dinv, state, conv_b, l1wt, l1b, l2wt, l2b,
            dirw, dirb):
    nblk = N // BLK
    blk = lambda cdim: pl.BlockSpec((BLK, cdim), lambda i: (i, 0))
    full = lambda rdim, cdim: pl.BlockSpec((rdim, cdim), lambda i: (0, 0))
    return pl.pallas_call(
        _mlp_body,
        grid=(nblk,),
        in_specs=[
            blk(HALF), blk(HALF), blk(HALF), blk(HALF), blk(1), blk(D),
            full(1, D), full(D, D), full(1, D), full(D, D), full(1, D),
            full(1, D), full(1, 1),
        ],
        out_specs=blk(1),
        out_shape=jax.ShapeDtypeStruct((N, 1), jnp.float32),
    )(agg0, agg1, y0, y1, dinv, state, conv_b, l1wt, l1b, l2wt, l2b, dirw,
      dirb)


def _norm_body(c_ref, o_ref):
    c = c_ref[...]
    o_ref[...] = (c / (jnp.sum(c) + 1e-20)).reshape(N // ACT, ACT)


@jax.jit
def _tc_norm(conc):
    return pl.pallas_call(
        _norm_body,
        out_shape=jax.ShapeDtypeStruct((N // ACT, ACT), jnp.float32),
    )(conc)


def kernel(state, edge_index, deterministic, conv_W, conv_b, lin1_W, lin1_b,
           lin2_W, lin2_b, dir_W, dir_b):
    src = edge_index[0]
    dst = edge_index[1]
    src_agg = src.reshape(AGG_ROWS, AGG_CH)
    dst_agg = dst.reshape(AGG_ROWS, AGG_CH)
    ones128 = jnp.ones((128,), jnp.float32)
    zeros_deg = jnp.zeros((NPAD,), jnp.float32)

    deg0, deg1 = _sc_degree(dst_agg, ones128, zeros_deg)
    xw = _tc_xw(state, conv_W.T)
    y0, y1, dinv = _tc_scale(xw, deg0[:N].reshape(N, 1),
                             deg1[:N].reshape(N, 1))
    agg0, agg1 = _sc_aggregate(y0, y1, src_agg, dst_agg)
    conc = _tc_mlp(agg0, agg1, y0, y1, dinv, state,
                   conv_b.reshape(1, D), lin1_W.T, lin1_b.reshape(1, D),
                   lin2_W.T, lin2_b.reshape(1, D), dir_W.reshape(1, D),
                   dir_b.reshape(1, 1))
    return _tc_norm(conc)


# speedup vs baseline: 20.2072x; 1.0572x over previous
import functools

import jax
import jax.numpy as jnp
from jax import lax
from jax.experimental import pallas as pl
from jax.experimental.pallas import tpu as pltpu
from jax.experimental.pallas import tpu_sc as plsc

N = 10000
E = 160000
D = 256
HALF = 128
ACT = 8

NPAD = 10240
ROWS_PER_TILE = NPAD // 16
BLK = 1024
NBLK = NPAD // BLK

AGG_CH = 125
AGG_ROWS = E // AGG_CH
AGG_TROWS = AGG_ROWS // 16
DEG_TROWS = AGG_ROWS // 32

ZROWS = 32


def _sc_mesh():
    return plsc.VectorSubcoreMesh(core_axis_name="c", subcore_axis_name="s")


def _deg_body(dst_hbm, ones_hbm, zeros_hbm, deg0_hbm, deg1_hbm,
              dstv, onesv, accd):
    c = lax.axis_index("c")
    s = lax.axis_index("s")
    pltpu.sync_copy(zeros_hbm.at[pl.ds(s * 640, 640)],
                    accd.at[pl.ds(s * 640, 640)])
    pltpu.sync_copy(ones_hbm.at[pl.ds(0, AGG_CH)], onesv)
    r0 = c * (AGG_ROWS // 2) + s * DEG_TROWS
    pltpu.sync_copy(dst_hbm.at[pl.ds(r0, DEG_TROWS)], dstv)
    plsc.subcore_barrier()

    def body(j, carry):
        pltpu.sync_copy(onesv, accd.at[dstv.at[j]], add=True)
        return carry

    lax.fori_loop(0, DEG_TROWS, body, 0)
    plsc.subcore_barrier()

    @pl.when(c == 0)
    def _():
        pltpu.sync_copy(accd.at[pl.ds(s * 640, 640)],
                        deg0_hbm.at[pl.ds(s * 640, 640)])

    @pl.when(c == 1)
    def _():
        pltpu.sync_copy(accd.at[pl.ds(s * 640, 640)],
                        deg1_hbm.at[pl.ds(s * 640, 640)])


@jax.jit
def _sc_degree(dst_a, ones_hbm, zeros_deg):
    f = functools.partial(
        pl.kernel,
        mesh=_sc_mesh(),
        out_type=[jax.ShapeDtypeStruct((NPAD,), jnp.float32),
                  jax.ShapeDtypeStruct((NPAD,), jnp.float32)],
        scratch_types=[
            pltpu.VMEM((DEG_TROWS, AGG_CH), jnp.int32),
            pltpu.VMEM((AGG_CH,), jnp.float32),
            pltpu.VMEM_SHARED((NPAD,), jnp.float32),
        ],
    )
    return f(_deg_body)(dst_a, ones_hbm, zeros_deg)


def _agg_body(y0_hbm, y1_hbm, src_hbm, dst_hbm,
              agg0_hbm, agg1_hbm, srcv, dstv, rows0, rows1, zbuf, acc,
              sem0, sem1):
    c = lax.axis_index("c")
    s = lax.axis_index("s")

    def zstore(r, carry):
        for k in range(HALF // 16):
            zbuf[r, pl.ds(16 * k, 16)] = jnp.zeros((16,), jnp.float32)
        return carry

    lax.fori_loop(0, ZROWS, zstore, 0)
    for m in range(ROWS_PER_TILE // ZROWS):
        pltpu.sync_copy(zbuf,
                        acc.at[pl.ds(s * ROWS_PER_TILE + m * ZROWS, ZROWS)])

    r0 = s * AGG_TROWS
    plsc.subcore_barrier()

    HTR = AGG_TROWS // 2

    def make_loop(ytab):
        def start(j, buf, sem):
            pltpu.async_copy(ytab.at[srcv.at[j]], buf, sem)

        def drain(j, buf, sem):
            pltpu.make_async_copy(ytab.at[srcv.at[j]], buf, sem).wait()

        for h in range(2):
            pltpu.sync_copy(src_hbm.at[pl.ds(r0 + h * HTR, HTR)], srcv)
            pltpu.sync_copy(dst_hbm.at[pl.ds(r0 + h * HTR, HTR)], dstv)
            start(0, rows0, sem0)

            def body(k, carry):
                j0 = 2 * k
                j1 = j0 + 1
                drain(j0, rows0, sem0)
                start(j1, rows1, sem1)
                pltpu.sync_copy(rows0, acc.at[dstv.at[j0]], add=True)
                drain(j1, rows1, sem1)

                @pl.when(k < HTR // 2 - 1)
                def _():
                    start(j1 + 1, rows0, sem0)

                pltpu.sync_copy(rows1, acc.at[dstv.at[j1]], add=True)
                return carry

            lax.fori_loop(0, HTR // 2, body, 0)

    @pl.when(c == 0)
    def _():
        make_loop(y0_hbm)

    @pl.when(c == 1)
    def _():
        make_loop(y1_hbm)

    plsc.subcore_barrier()

    @pl.when(c == 0)
    def _():
        pltpu.sync_copy(acc.at[pl.ds(s * ROWS_PER_TILE, ROWS_PER_TILE)],
                        agg0_hbm.at[pl.ds(s * ROWS_PER_TILE, ROWS_PER_TILE)])

    @pl.when(c == 1)
    def _():
        pltpu.sync_copy(acc.at[pl.ds(s * ROWS_PER_TILE, ROWS_PER_TILE)],
                        agg1_hbm.at[pl.ds(s * ROWS_PER_TILE, ROWS_PER_TILE)])


@jax.jit
def _sc_aggregate(y0, y1, src_a, dst_a):
    f = functools.partial(
        pl.kernel,
        mesh=_sc_mesh(),
        out_type=[jax.ShapeDtypeStruct((NPAD, HALF), jnp.float32),
                  jax.ShapeDtypeStruct((NPAD, HALF), jnp.float32)],
        scratch_types=[
            pltpu.VMEM((AGG_TROWS // 2, AGG_CH), jnp.int32),
            pltpu.VMEM((AGG_TROWS // 2, AGG_CH), jnp.int32),
            pltpu.VMEM((AGG_CH, HALF), jnp.float32),
            pltpu.VMEM((AGG_CH, HALF), jnp.float32),
            pltpu.VMEM((ZROWS, HALF), jnp.float32),
            pltpu.VMEM_SHARED((NPAD, HALF), jnp.float32),
            pltpu.SemaphoreType.DMA,
            pltpu.SemaphoreType.DMA,
        ],
    )
    return f(_agg_body)(y0, y1, src_a, dst_a)


def _xw_body(x_ref, wt_ref, xw_ref):
    xw_ref[...] = jnp.dot(x_ref[...], wt_ref[...],
                          preferred_element_type=jnp.float32)


@jax.jit
def _tc_xw(state, conv_Wt):
    return pl.pallas_call(
        _xw_body,
        grid=(NBLK,),
        in_specs=[
            pl.BlockSpec((BLK, D), lambda i: (i, 0)),
            pl.BlockSpec((D, D), lambda i: (0, 0)),
        ],
        out_specs=pl.BlockSpec((BLK, D), lambda i: (i, 0)),
        out_shape=jax.ShapeDtypeStruct((NPAD, D), jnp.float32),
    )(state, conv_Wt)


def _scale_body(xw_ref, d0_ref, d1_ref, y0_ref, y1_ref, dinv_ref):
    deg = d0_ref[...] + d1_ref[...] + 1.0
    dinv = lax.rsqrt(deg).reshape(BLK, 1)
    y = xw_ref[...] * dinv
    y0_ref[...] = y[:, :HALF]
    y1_ref[...] = y[:, HALF:]
    dinv_ref[...] = dinv


@jax.jit
def _tc_scale(xw, deg0, deg1):
    return pl.pallas_call(
        _scale_body,
        grid=(NBLK,),
        in_specs=[
            pl.BlockSpec((BLK, D), lambda i: (i, 0)),
            pl.BlockSpec((BLK,), lambda i: (i,)),
            pl.BlockSpec((BLK,), lambda i: (i,)),
        ],
        out_specs=[
            pl.BlockSpec((BLK, HALF), lambda i: (i, 0)),
            pl.BlockSpec((BLK, HALF), lambda i: (i, 0)),
            pl.BlockSpec((BLK, 1), lambda i: (i, 0)),
        ],
        out_shape=[
            jax.ShapeDtypeStruct((NPAD, HALF), jnp.float32),
            jax.ShapeDtypeStruct((NPAD, HALF), jnp.float32),
            jax.ShapeDtypeStruct((NPAD, 1), jnp.float32),
        ],
    )(xw, deg0, deg1)


def _mlp_body(a0_ref, a1_ref, y0_ref, y1_ref, dinv_ref, st_ref, cb_ref,
              w1_ref, b1_ref, w2_ref, b2_ref, dw_ref, db_ref, conc_ref):
    i = pl.program_id(0)
    agg = jnp.concatenate([a0_ref[...], a1_ref[...]], axis=1)
    y = jnp.concatenate([y0_ref[...], y1_ref[...]], axis=1)
    dinv = dinv_ref[...]
    x = jnp.maximum(dinv * (agg + y) + cb_ref[...], 0.0) + st_ref[...]
    h = jnp.dot(x, w1_ref[...], preferred_element_type=jnp.float32) + b1_ref[...]
    h = jnp.where(h > 0, h, 0.01 * h)
    h = jnp.dot(h, w2_ref[...], preferred_element_type=jnp.float32) + b2_ref[...]
    h = jnp.where(h > 0, h, 0.01 * h)
    z = jnp.sum(h * dw_ref[...], axis=1, keepdims=True) + db_ref[...]
    sp = jnp.maximum(z, 0.0) + jnp.log1p(jnp.exp(-jnp.abs(z)))
    rid = i * BLK + lax.broadcasted_iota(jnp.int32, (BLK, 1), 0)
    conc_ref[...] = jnp.where(rid < N, sp, 0.0)


@jax.jit
def _tc_mlp(agg0, agg1, y0, y1, dinv, state, conv_b, l1wt, l1b, l2wt, l2b,
            dirw, dirb):
    blk = lambda cdim: pl.BlockSpec((BLK, cdim), lambda i: (i, 0))
    full = lambda rdim, cdim: pl.BlockSpec((rdim, cdim), lambda i: (0, 0))
    return pl.pallas_call(
        _mlp_body,
        grid=(NBLK,),
        in_specs=[
            blk(HALF), blk(HALF), blk(HALF), blk(HALF), blk(1), blk(D),
            full(1, D), full(D, D), full(1, D), full(D, D), full(1, D),
            full(1, D), full(1, 1),
        ],
        out_specs=blk(1),
        out_shape=jax.ShapeDtypeStruct((NPAD, 1), jnp.float32),
    )(agg0, agg1, y0, y1, dinv, state, conv_b, l1wt, l1b, l2wt, l2b, dirw,
      dirb)


def _norm_body(c_ref, o_ref):
    c = c_ref[...]
    o_ref[...] = (c[:N] / (jnp.sum(c) + 1e-20)).reshape(N // ACT, ACT)


@jax.jit
def _tc_norm(conc):
    return pl.pallas_call(
        _norm_body,
        out_shape=jax.ShapeDtypeStruct((N // ACT, ACT), jnp.float32),
    )(conc)


def kernel(state, edge_index, deterministic, conv_W, conv_b, lin1_W, lin1_b,
           lin2_W, lin2_b, dir_W, dir_b):
    src = edge_index[0]
    dst = edge_index[1]
    src_agg = src.reshape(AGG_ROWS, AGG_CH)
    dst_agg = dst.reshape(AGG_ROWS, AGG_CH)
    ones128 = jnp.ones((128,), jnp.float32)
    zeros_deg = jnp.zeros((NPAD,), jnp.float32)

    deg0, deg1 = _sc_degree(dst_agg, ones128, zeros_deg)
    xw = _tc_xw(state, conv_W.T)
    y0, y1, dinv = _tc_scale(xw, deg0, deg1)
    agg0, agg1 = _sc_aggregate(y0, y1, src_agg, dst_agg)
    conc = _tc_mlp(agg0, agg1, y0, y1, dinv, state,
                   conv_b.reshape(1, D), lin1_W.T, lin1_b.reshape(1, D),
                   lin2_W.T, lin2_b.reshape(1, D), dir_W.reshape(1, D),
                   dir_b.reshape(1, 1))
    return _tc_norm(conc)


# speedup vs baseline: 20.3020x; 1.0047x over previous
import functools

import jax
import jax.numpy as jnp
from jax import lax
from jax.experimental import pallas as pl
from jax.experimental.pallas import tpu as pltpu
from jax.experimental.pallas import tpu_sc as plsc

N = 10000
E = 160000
D = 256
HALF = 128
ACT = 8

NPAD = 10240
ROWS_PER_TILE = NPAD // 16
BLK = 1024
NBLK = NPAD // BLK

AGG_CH = 125
AGG_ROWS = E // AGG_CH
AGG_TROWS = AGG_ROWS // 16
DEG_TROWS = AGG_ROWS // 32

ZROWS = 32


def _sc_mesh():
    return plsc.VectorSubcoreMesh(core_axis_name="c", subcore_axis_name="s")


def _deg_body(dst_hbm, ones_hbm, zeros_hbm, deg0_hbm, deg1_hbm,
              dstv, onesv, accd):
    c = lax.axis_index("c")
    s = lax.axis_index("s")
    pltpu.sync_copy(zeros_hbm.at[pl.ds(s * 640, 640)],
                    accd.at[pl.ds(s * 640, 640)])
    pltpu.sync_copy(ones_hbm.at[pl.ds(0, AGG_CH)], onesv)
    r0 = c * (AGG_ROWS // 2) + s * DEG_TROWS
    pltpu.sync_copy(dst_hbm.at[pl.ds(r0, DEG_TROWS)], dstv)
    plsc.subcore_barrier()

    def body(j, carry):
        pltpu.sync_copy(onesv, accd.at[dstv.at[j]], add=True)
        return carry

    lax.fori_loop(0, DEG_TROWS, body, 0)
    plsc.subcore_barrier()

    @pl.when(c == 0)
    def _():
        pltpu.sync_copy(accd.at[pl.ds(s * 640, 640)],
                        deg0_hbm.at[pl.ds(s * 640, 640)])

    @pl.when(c == 1)
    def _():
        pltpu.sync_copy(accd.at[pl.ds(s * 640, 640)],
                        deg1_hbm.at[pl.ds(s * 640, 640)])


@jax.jit
def _sc_degree(dst_a, ones_hbm, zeros_deg):
    f = functools.partial(
        pl.kernel,
        mesh=_sc_mesh(),
        out_type=[jax.ShapeDtypeStruct((NPAD,), jnp.float32),
                  jax.ShapeDtypeStruct((NPAD,), jnp.float32)],
        scratch_types=[
            pltpu.VMEM((DEG_TROWS, AGG_CH), jnp.int32),
            pltpu.VMEM((AGG_CH,), jnp.float32),
            pltpu.VMEM_SHARED((NPAD,), jnp.float32),
        ],
    )
    return f(_deg_body)(dst_a, ones_hbm, zeros_deg)


def _agg_body(y0_hbm, y1_hbm, src_hbm, dst_hbm,
              agg0_hbm, agg1_hbm, srcv, dstv, rows0, rows1, acc,
              sem0, sem1):
    c = lax.axis_index("c")
    s = lax.axis_index("s")

    @pl.when(c == 0)
    def _():
        pltpu.sync_copy(y0_hbm.at[pl.ds(s * ROWS_PER_TILE, ROWS_PER_TILE)],
                        acc.at[pl.ds(s * ROWS_PER_TILE, ROWS_PER_TILE)])

    @pl.when(c == 1)
    def _():
        pltpu.sync_copy(y1_hbm.at[pl.ds(s * ROWS_PER_TILE, ROWS_PER_TILE)],
                        acc.at[pl.ds(s * ROWS_PER_TILE, ROWS_PER_TILE)])

    r0 = s * AGG_TROWS
    plsc.subcore_barrier()

    HTR = AGG_TROWS // 2

    def make_loop(ytab):
        def start(j, buf, sem):
            pltpu.async_copy(ytab.at[srcv.at[j]], buf, sem)

        def drain(j, buf, sem):
            pltpu.make_async_copy(ytab.at[srcv.at[j]], buf, sem).wait()

        for h in range(2):
            pltpu.sync_copy(src_hbm.at[pl.ds(r0 + h * HTR, HTR)], srcv)
            pltpu.sync_copy(dst_hbm.at[pl.ds(r0 + h * HTR, HTR)], dstv)
            start(0, rows0, sem0)

            def body(k, carry):
                j0 = 2 * k
                j1 = j0 + 1
                drain(j0, rows0, sem0)
                start(j1, rows1, sem1)
                pltpu.sync_copy(rows0, acc.at[dstv.at[j0]], add=True)
                drain(j1, rows1, sem1)

                @pl.when(k < HTR // 2 - 1)
                def _():
                    start(j1 + 1, rows0, sem0)

                pltpu.sync_copy(rows1, acc.at[dstv.at[j1]], add=True)
                return carry

            lax.fori_loop(0, HTR // 2, body, 0)

    @pl.when(c == 0)
    def _():
        make_loop(y0_hbm)

    @pl.when(c == 1)
    def _():
        make_loop(y1_hbm)

    plsc.subcore_barrier()

    @pl.when(c == 0)
    def _():
        pltpu.sync_copy(acc.at[pl.ds(s * ROWS_PER_TILE, ROWS_PER_TILE)],
                        agg0_hbm.at[pl.ds(s * ROWS_PER_TILE, ROWS_PER_TILE)])

    @pl.when(c == 1)
    def _():
        pltpu.sync_copy(acc.at[pl.ds(s * ROWS_PER_TILE, ROWS_PER_TILE)],
                        agg1_hbm.at[pl.ds(s * ROWS_PER_TILE, ROWS_PER_TILE)])


@jax.jit
def _sc_aggregate(y0, y1, src_a, dst_a):
    f = functools.partial(
        pl.kernel,
        mesh=_sc_mesh(),
        out_type=[jax.ShapeDtypeStruct((NPAD, HALF), jnp.float32),
                  jax.ShapeDtypeStruct((NPAD, HALF), jnp.float32)],
        scratch_types=[
            pltpu.VMEM((AGG_TROWS // 2, AGG_CH), jnp.int32),
            pltpu.VMEM((AGG_TROWS // 2, AGG_CH), jnp.int32),
            pltpu.VMEM((AGG_CH, HALF), jnp.float32),
            pltpu.VMEM((AGG_CH, HALF), jnp.float32),
            pltpu.VMEM_SHARED((NPAD, HALF), jnp.float32),
            pltpu.SemaphoreType.DMA,
            pltpu.SemaphoreType.DMA,
        ],
    )
    return f(_agg_body)(y0, y1, src_a, dst_a)


def _xw_body(x_ref, wt_ref, xw_ref):
    xw_ref[...] = jnp.dot(x_ref[...], wt_ref[...],
                          preferred_element_type=jnp.float32)


@jax.jit
def _tc_xw(state, conv_Wt):
    return pl.pallas_call(
        _xw_body,
        grid=(NBLK,),
        in_specs=[
            pl.BlockSpec((BLK, D), lambda i: (i, 0)),
            pl.BlockSpec((D, D), lambda i: (0, 0)),
        ],
        out_specs=pl.BlockSpec((BLK, D), lambda i: (i, 0)),
        out_shape=jax.ShapeDtypeStruct((NPAD, D), jnp.float32),
    )(state, conv_Wt)


def _scale_body(xw_ref, d0_ref, d1_ref, y0_ref, y1_ref, dinv_ref):
    deg = d0_ref[...] + d1_ref[...] + 1.0
    dinv = lax.rsqrt(deg)
    y = xw_ref[...] * dinv.reshape(BLK, 1)
    y0_ref[...] = y[:, :HALF]
    y1_ref[...] = y[:, HALF:]
    dinv_ref[...] = dinv


@jax.jit
def _tc_scale(xw, deg0, deg1):
    return pl.pallas_call(
        _scale_body,
        grid=(NBLK,),
        in_specs=[
            pl.BlockSpec((BLK, D), lambda i: (i, 0)),
            pl.BlockSpec((BLK,), lambda i: (i,)),
            pl.BlockSpec((BLK,), lambda i: (i,)),
        ],
        out_specs=[
            pl.BlockSpec((BLK, HALF), lambda i: (i, 0)),
            pl.BlockSpec((BLK, HALF), lambda i: (i, 0)),
            pl.BlockSpec((BLK,), lambda i: (i,)),
        ],
        out_shape=[
            jax.ShapeDtypeStruct((NPAD, HALF), jnp.float32),
            jax.ShapeDtypeStruct((NPAD, HALF), jnp.float32),
            jax.ShapeDtypeStruct((NPAD,), jnp.float32),
        ],
    )(xw, deg0, deg1)


def _mlp_body(a0_ref, a1_ref, dinv_ref, st_ref, cb_ref,
              w1_ref, b1_ref, w2_ref, b2_ref, dw_ref, db_ref, conc_ref):
    i = pl.program_id(0)
    agg = jnp.concatenate([a0_ref[...], a1_ref[...]], axis=1)
    dinv = dinv_ref[...].reshape(BLK, 1)
    x = jnp.maximum(dinv * agg + cb_ref[...], 0.0) + st_ref[...]
    h = jnp.dot(x, w1_ref[...], preferred_element_type=jnp.float32) + b1_ref[...]
    h = jnp.where(h > 0, h, 0.01 * h)
    h = jnp.dot(h, w2_ref[...], preferred_element_type=jnp.float32) + b2_ref[...]
    h = jnp.where(h > 0, h, 0.01 * h)
    z = jnp.sum(h * dw_ref[...], axis=1, keepdims=True) + db_ref[...]
    sp = jnp.maximum(z, 0.0) + jnp.log1p(jnp.exp(-jnp.abs(z)))
    rid = i * BLK + lax.broadcasted_iota(jnp.int32, (BLK, 1), 0)
    conc_ref[...] = jnp.where(rid < N, sp, 0.0)


@jax.jit
def _tc_mlp(agg0, agg1, dinv, state, conv_b, l1wt, l1b, l2wt, l2b,
            dirw, dirb):
    blk = lambda cdim: pl.BlockSpec((BLK, cdim), lambda i: (i, 0))
    full = lambda rdim, cdim: pl.BlockSpec((rdim, cdim), lambda i: (0, 0))
    return pl.pallas_call(
        _mlp_body,
        grid=(NBLK,),
        in_specs=[
            blk(HALF), blk(HALF), pl.BlockSpec((BLK,), lambda i: (i,)),
            blk(D),
            full(1, D), full(D, D), full(1, D), full(D, D), full(1, D),
            full(1, D), full(1, 1),
        ],
        out_specs=blk(1),
        out_shape=jax.ShapeDtypeStruct((NPAD, 1), jnp.float32),
    )(agg0, agg1, dinv, state, conv_b, l1wt, l1b, l2wt, l2b, dirw, dirb)


def _norm_body(c_ref, o_ref):
    c = c_ref[...]
    o_ref[...] = (c[:N] / (jnp.sum(c) + 1e-20)).reshape(N // ACT, ACT)


@jax.jit
def _tc_norm(conc):
    return pl.pallas_call(
        _norm_body,
        out_shape=jax.ShapeDtypeStruct((N // ACT, ACT), jnp.float32),
    )(conc)


def kernel(state, edge_index, deterministic, conv_W, conv_b, lin1_W, lin1_b,
           lin2_W, lin2_b, dir_W, dir_b):
    src = edge_index[0]
    dst = edge_index[1]
    src_agg = src.reshape(AGG_ROWS, AGG_CH)
    dst_agg = dst.reshape(AGG_ROWS, AGG_CH)
    ones128 = jnp.ones((128,), jnp.float32)
    zeros_deg = jnp.zeros((NPAD,), jnp.float32)

    deg0, deg1 = _sc_degree(dst_agg, ones128, zeros_deg)
    xw = _tc_xw(state, conv_W.T)
    y0, y1, dinv = _tc_scale(xw, deg0, deg1)
    agg0, agg1 = _sc_aggregate(y0, y1, src_agg, dst_agg)
    conc = _tc_mlp(agg0, agg1, dinv, state,
                   conv_b.reshape(1, D), lin1_W.T, lin1_b.reshape(1, D),
                   lin2_W.T, lin2_b.reshape(1, D), dir_W.reshape(1, D),
                   dir_b.reshape(1, 1))
    return _tc_norm(conc)
